# baseline (speedup 1.0000x reference)
import functools

import jax
import jax.numpy as jnp
from jax import lax
from jax.experimental import pallas as pl
from jax.experimental.pallas import tpu as pltpu
from jax.experimental.pallas import tpu_sc as plsc

B = 4096
D0, D1, D2 = 1024, 1024, 2048
F = 2048
E = 8
TOPK = 2
A = B * TOPK
BLK = 256
NR = A + E * BLK
NBLK = NR // BLK
NS = 16
NC = 2
SEG = A // NS
ZSEG = NR // NS
GROWS = NR // (NS * NC)
GCH = 16
TB = B // (NS * NC)
CC = 16

_INV_SQRT2 = 0.7071067811865476


def _gelu_exact(x):
    return x * (0.5 * (1.0 + jax.lax.erf(x * _INV_SQRT2)))


def _dot(a, b):
    return jax.lax.dot_general(
        a, b, (((1,), (0,)), ((), ())), preferred_element_type=jnp.float32)


BM_A = 512
BN_A = 512


def _shared_body(f0, f1, f2, w, b, wg, bg, h32, ti, tw, logit_acc):
    j = pl.program_id(1)
    acc = _dot(f0[...], w[0:D0, :])
    acc += _dot(f1[...], w[D0:D0 + D1, :])
    acc += _dot(f2[...], w[D0 + D1:D0 + D1 + D2, :])
    acc = acc + b[...]
    h = _gelu_exact(acc)
    h32[...] = h
    lg = _dot(h.astype(jnp.bfloat16), wg[...])

    @pl.when(j == 0)
    def _():
        logit_acc[...] = lg

    @pl.when(j != 0)
    def _():
        logit_acc[...] += lg

    @pl.when(j == pl.num_programs(1) - 1)
    def _():
        logits = logit_acc[...] + bg[...]
        m = jnp.max(logits, axis=1, keepdims=True)
        ex = jnp.exp(logits - m)
        p = ex / jnp.sum(ex, axis=1, keepdims=True)
        lane = jax.lax.broadcasted_iota(jnp.int32, p.shape, 1)
        i1 = jnp.argmax(p, axis=1)[:, None].astype(jnp.int32)
        w1 = jnp.max(p, axis=1, keepdims=True)
        p2 = jnp.where(lane == i1, -1.0, p)
        i2 = jnp.argmax(p2, axis=1)[:, None].astype(jnp.int32)
        w2 = jnp.max(p2, axis=1, keepdims=True)
        ti[...] = jnp.concatenate([i1, i2], axis=1)
        tw[...] = jnp.concatenate([w1, w2], axis=1)


def _shared_proj(f0, f1, f2, W16, b_shared, Wg16, b_gate):
    grid = (B // BM_A, F // BN_A)
    return pl.pallas_call(
        _shared_body,
        grid=grid,
        in_specs=[
            pl.BlockSpec((BM_A, D0), lambda i, j: (i, 0)),
            pl.BlockSpec((BM_A, D1), lambda i, j: (i, 0)),
            pl.BlockSpec((BM_A, D2), lambda i, j: (i, 0)),
            pl.BlockSpec((D0 + D1 + D2, BN_A), lambda i, j: (0, j)),
            pl.BlockSpec((1, BN_A), lambda i, j: (0, j)),
            pl.BlockSpec((BN_A, E), lambda i, j: (j, 0)),
            pl.BlockSpec((1, E), lambda i, j: (0, 0)),
        ],
        out_specs=[
            pl.BlockSpec((BM_A, BN_A), lambda i, j: (i, j)),
            pl.BlockSpec((BM_A, TOPK), lambda i, j: (i, 0)),
            pl.BlockSpec((BM_A, TOPK), lambda i, j: (i, 0)),
        ],
        out_shape=[
            jax.ShapeDtypeStruct((B, F), jnp.float32),
            jax.ShapeDtypeStruct((B, TOPK), jnp.int32),
            jax.ShapeDtypeStruct((B, TOPK), jnp.float32),
        ],
        scratch_shapes=[pltpu.VMEM((BM_A, E), jnp.float32)],
        compiler_params=pltpu.CompilerParams(
            dimension_semantics=("parallel", "arbitrary")),
    )(f0, f1, f2, W16, b_shared.reshape(1, F), Wg16, b_gate.reshape(1, E))


@functools.cache
def _route_gather_kernel():
    mesh = plsc.VectorSubcoreMesh(
        core_axis_name="c", subcore_axis_name="s",
        num_cores=NC, num_subcores=NS)

    @functools.partial(
        pl.kernel,
        out_type=[
            jax.ShapeDtypeStruct((NR, F), jnp.float32),
            jax.ShapeDtypeStruct((NR,), jnp.float32),
            jax.ShapeDtypeStruct((A,), jnp.int32),
            jax.ShapeDtypeStruct((64,), jnp.int32),
        ],
        mesh=mesh,
        scratch_types=[
            pltpu.VMEM((SEG,), jnp.int32),
            pltpu.VMEM((SEG,), jnp.float32),
            pltpu.VMEM((SEG,), jnp.int32),
            pltpu.VMEM((16,), jnp.int32),
            pltpu.VMEM((16, 16), jnp.int32),
            pltpu.VMEM((NR,), jnp.int32),
            pltpu.VMEM((NR,), jnp.float32),
            pltpu.VMEM((ZSEG,), jnp.int32),
            pltpu.VMEM((ZSEG,), jnp.int32),
            pltpu.VMEM((ZSEG,), jnp.float32),
            pltpu.VMEM((ZSEG,), jnp.float32),
            pltpu.VMEM((64,), jnp.int32),
            pltpu.VMEM((GROWS,), jnp.int32),
            pltpu.VMEM((GCH, F), jnp.float32),
            pltpu.VMEM((GCH, F), jnp.float32),
            pltpu.VMEM_SHARED((16, 16), jnp.int32),
            pltpu.VMEM_SHARED((NS, NR), jnp.int32),
            pltpu.VMEM_SHARED((NS, NR), jnp.float32),
            pltpu.VMEM_SHARED((NR,), jnp.int32),
            pltpu.VMEM_SHARED((NR,), jnp.float32),
            pltpu.SemaphoreType.DMA,
            pltpu.SemaphoreType.DMA,
            pltpu.SemaphoreType.DMA,
            pltpu.SemaphoreType.DMA,
        ],
        compiler_params=pltpu.CompilerParams(needs_layout_passes=False),
    )
    def route_gather(tik, twk, h32, hg, wrow_o, pos_o, gid_o,
                     ti_v, tw_v, pos_v, cnt_st, cnt_all, nrbuf_i, nrbuf_f,
                     tmp_i, acc_i, tmp_f, acc_f, gid_v, src_local, bufA, bufB,
                     cnt_sh, all_i, all_f, src_sh, wrow_sh,
                     semG0, semG1, semW0, semW1):
        s = lax.axis_index("s")
        c = lax.axis_index("c")
        lane = lax.iota(jnp.int32, 16)
        zi16 = jnp.zeros((16,), jnp.int32)
        zf16 = jnp.zeros((16,), jnp.float32)

        def bc(x):
            return jnp.broadcast_to(jnp.asarray(x, jnp.int32), (16,))

        pltpu.sync_copy(tik.at[pl.ds(s * SEG, SEG)], ti_v)
        pltpu.sync_copy(twk.at[pl.ds(s * SEG, SEG)], tw_v)

        def _zero(i, _):
            nrbuf_i[pl.ds(i * 16, 16)] = zi16
            nrbuf_f[pl.ds(i * 16, 16)] = zf16
            return 0
        lax.fori_loop(0, NR // 16, _zero, 0)

        cnt = zi16
        for i in range(SEG // 16):
            v = ti_v[pl.ds(i * 16, 16)]
            for e in range(E):
                pc = jnp.sum((v == bc(e)).astype(jnp.int32))
                cnt = cnt + jnp.where(lane == bc(e), bc(pc), zi16)
        cnt_st[...] = cnt
        pltpu.sync_copy(cnt_st, cnt_sh.at[s])
        plsc.subcore_barrier()

        pltpu.sync_copy(cnt_sh, cnt_all)
        pre = zi16
        tot = zi16
        for w in range(NS):
            row = cnt_all[w]
            tot = tot + row
            wmask = bc((jnp.int32(w) < s).astype(jnp.int32))
            pre = pre + wmask * row
        padded = (tot + bc(BLK - 1)) & bc(~(BLK - 1))
        padded = jnp.where(lane < bc(E), padded, zi16)
        incl = plsc.cumsum(padded)
        excl = incl - padded
        base = excl + pre
        total_padded = jnp.sum(jnp.where(lane == bc(E - 1), incl, zi16))
        nb = total_padded // BLK
        exc_s = [jnp.sum(jnp.where(lane == bc(e), excl, zi16)) for e in range(E)]
        pad_s = [jnp.sum(jnp.where(lane == bc(e), padded, zi16)) for e in range(E)]

        cur = base
        for i in range(SEG // 16):
            v = ti_v[pl.ds(i * 16, 16)]
            twv = tw_v[pl.ds(i * 16, 16)]
            avec = bc(s * SEG + i * 16) + lane
            tok = avec & bc(B - 1)
            posv = zi16
            for e in range(E):
                mi = (v == bc(e)).astype(jnp.int32)
                pm = plsc.cumsum(mi)
                curE = jnp.sum(jnp.where(lane == bc(e), cur, zi16))
                posv = posv + mi * (bc(curE) + pm - bc(1))
                pc = jnp.sum(mi)
                cur = cur + jnp.where(lane == bc(e), bc(pc), zi16)
            pos_v[pl.ds(i * 16, 16)] = posv
            plsc.store_scatter(nrbuf_i, [posv], tok)
            plsc.store_scatter(nrbuf_f, [posv], twv)

        pltpu.sync_copy(nrbuf_i, all_i.at[s])
        pltpu.sync_copy(nrbuf_f, all_f.at[s])
        plsc.subcore_barrier()

        def _zacc(i, _):
            acc_i[pl.ds(i * 16, 16)] = zi16
            acc_f[pl.ds(i * 16, 16)] = zf16
            return 0
        lax.fori_loop(0, ZSEG // 16, _zacc, 0)

        def _merge(w, _):
            pltpu.sync_copy(all_i.at[w, pl.ds(s * ZSEG, ZSEG)], tmp_i)
            pltpu.sync_copy(all_f.at[w, pl.ds(s * ZSEG, ZSEG)], tmp_f)

            def _add(i, _):
                acc_i[pl.ds(i * 16, 16)] += tmp_i[pl.ds(i * 16, 16)]
                acc_f[pl.ds(i * 16, 16)] += tmp_f[pl.ds(i * 16, 16)]
                return 0
            lax.fori_loop(0, ZSEG // 16, _add, 0)
            return 0
        lax.fori_loop(0, NS, _merge, 0)
        pltpu.sync_copy(acc_i, src_sh.at[pl.ds(s * ZSEG, ZSEG)])
        pltpu.sync_copy(acc_f, wrow_sh.at[pl.ds(s * ZSEG, ZSEG)])
        plsc.subcore_barrier()

        @pl.when(c == 0)
        def _():
            pltpu.sync_copy(pos_v, pos_o.at[pl.ds(s * SEG, SEG)])
            pltpu.sync_copy(acc_f, wrow_o.at[pl.ds(s * ZSEG, ZSEG)])

        @pl.when(jnp.logical_and(c == 0, s == 0))
        def _():
            for vb in range(4):
                jb = (lane + bc(vb * 16)) * bc(BLK)
                g = zi16
                for e in range(E):
                    inblk = jnp.logical_and(jb >= bc(exc_s[e]),
                                            jb < bc(exc_s[e] + pad_s[e]))
                    g = g + bc(e) * inblk.astype(jnp.int32)
                if vb == 3:
                    g = g + jnp.where(lane == bc(15), bc(nb), zi16)
                gid_v[pl.ds(vb * 16, 16)] = g
            pltpu.sync_copy(gid_v, gid_o)

        wid = s * NC + c
        gbase = wid * GROWS
        pltpu.sync_copy(src_sh.at[pl.ds(gbase, GROWS)], src_local)
        nch = GROWS // GCH
        bufs = [bufA, bufB]
        gsems = [semG0, semG1]
        wsems = [semW0, semW1]
        pltpu.async_copy(h32.at[src_local.at[pl.ds(0, GCH)]], bufs[0], gsems[0])
        for ch in range(nch):
            p = ch % 2
            pltpu.make_async_copy(h32.at[src_local.at[pl.ds(ch * GCH, GCH)]],
                                  bufs[p], gsems[p]).wait()
            if ch + 1 < nch:
                q = (ch + 1) % 2
                if ch - 1 >= 0:
                    pltpu.make_async_copy(
                        bufs[q], hg.at[pl.ds(gbase + (ch - 1) * GCH, GCH)],
                        wsems[q]).wait()
                pltpu.async_copy(h32.at[src_local.at[pl.ds((ch + 1) * GCH, GCH)]],
                                 bufs[q], gsems[q])
            pltpu.async_copy(bufs[p], hg.at[pl.ds(gbase + ch * GCH, GCH)],
                             wsems[p])
        pltpu.make_async_copy(bufs[(nch - 2) % 2],
                              hg.at[pl.ds(gbase + (nch - 2) * GCH, GCH)],
                              wsems[(nch - 2) % 2]).wait()
        pltpu.make_async_copy(bufs[(nch - 1) % 2],
                              hg.at[pl.ds(gbase + (nch - 1) * GCH, GCH)],
                              wsems[(nch - 1) % 2]).wait()

    return route_gather


def _grouped_body(gid_ref, hg, wexp, bexp, wr, y):
    j = pl.program_id(0)
    nb = gid_ref[63]

    @pl.when(j < nb)
    def _():
        acc = _dot(hg[...].astype(jnp.bfloat16), wexp[0]) + bexp[0]
        y[...] = wr[...] * _gelu_exact(acc)


def _grouped_experts(gid, hg, W16, b_exp3, wrow2):
    grid_spec = pltpu.PrefetchScalarGridSpec(
        num_scalar_prefetch=1,
        grid=(NBLK,),
        in_specs=[
            pl.BlockSpec((BLK, F), lambda j, g: (j, 0)),
            pl.BlockSpec((1, F, F), lambda j, g: (g[j], 0, 0)),
            pl.BlockSpec((1, 1, F), lambda j, g: (g[j], 0, 0)),
            pl.BlockSpec((BLK, 1), lambda j, g: (j, 0)),
        ],
        out_specs=pl.BlockSpec((BLK, F), lambda j, g: (j, 0)),
    )
    return pl.pallas_call(
        _grouped_body,
        grid_spec=grid_spec,
        out_shape=jax.ShapeDtypeStruct((NR, F), jnp.float32),
        compiler_params=pltpu.CompilerParams(
            dimension_semantics=("arbitrary",)),
    )(gid, hg, W16, b_exp3, wrow2)


@functools.cache
def _combine_kernel():
    mesh = plsc.VectorSubcoreMesh(
        core_axis_name="c", subcore_axis_name="s",
        num_cores=NC, num_subcores=NS)

    @functools.partial(
        pl.kernel,
        out_type=jax.ShapeDtypeStruct((B, F), jnp.float32),
        mesh=mesh,
        scratch_types=[
            pltpu.VMEM((TB,), jnp.int32),
            pltpu.VMEM((TB,), jnp.int32),
            pltpu.VMEM((CC, F), jnp.float32),
            pltpu.VMEM((CC, F), jnp.float32),
            pltpu.SemaphoreType.DMA,
            pltpu.SemaphoreType.DMA,
        ],
        compiler_params=pltpu.CompilerParams(needs_layout_passes=False),
    )
    def combine(pos_hbm, y_hbm, out_hbm, pos0_l, pos1_l, bufA, bufB,
                semA, semB):
        s = lax.axis_index("s")
        c = lax.axis_index("c")
        wid = s * NC + c
        t0 = wid * TB
        pltpu.sync_copy(pos_hbm.at[pl.ds(t0, TB)], pos0_l)
        pltpu.sync_copy(pos_hbm.at[pl.ds(B + t0, TB)], pos1_l)
        for ch in range(TB // CC):
            cpA = pltpu.async_copy(
                y_hbm.at[pos0_l.at[pl.ds(ch * CC, CC)]], bufA, semA)
            cpB = pltpu.async_copy(
                y_hbm.at[pos1_l.at[pl.ds(ch * CC, CC)]], bufB, semB)
            cpA.wait()
            cpB.wait()

            def _add(i, _):
                off = i * 16
                for r in range(CC):
                    bufA[r, pl.ds(off, 16)] += bufB[r, pl.ds(off, 16)]
                return 0
            lax.fori_loop(0, F // 16, _add, 0)
            pltpu.sync_copy(bufA, out_hbm.at[pl.ds(t0 + ch * CC, CC)])

    return combine


def kernel(feat0, feat1, feat2, W_shared, b_shared, W_gate, b_gate, W_exp, b_exp):
    bf = jnp.bfloat16
    h32, ti, tw = _shared_proj(feat0.astype(bf), feat1.astype(bf),
                               feat2.astype(bf), W_shared.astype(bf), b_shared,
                               W_gate.astype(bf), b_gate)
    tik = ti.T.reshape(A)
    twk = tw.T.reshape(A)
    hg, wrow, pos, gid = _route_gather_kernel()(tik, twk, h32)
    y = _grouped_experts(gid, hg, W_exp.astype(bf),
                         b_exp.reshape(E, 1, F), wrow.reshape(NR, 1))
    return _combine_kernel()(pos, y)


# speedup vs baseline: 1.1277x; 1.1277x over previous
import functools

import jax
import jax.numpy as jnp
from jax import lax
from jax.experimental import pallas as pl
from jax.experimental.pallas import tpu as pltpu
from jax.experimental.pallas import tpu_sc as plsc

B = 4096
D0, D1, D2 = 1024, 1024, 2048
F = 2048
E = 8
TOPK = 2
A = B * TOPK
BLK = 256
NR = A + E * BLK
NBLK = NR // BLK
NS = 16
NC = 2
SEG = A // NS
ZSEG = NR // NS
GROWS = NR // (NS * NC)
GCH = 16
TB = B // (NS * NC)
CC = 8

_INV_SQRT2 = 0.7071067811865476


def _gelu_exact(x):
    return x * (0.5 * (1.0 + jax.lax.erf(x * _INV_SQRT2)))


def _dot(a, b):
    return jax.lax.dot_general(
        a, b, (((1,), (0,)), ((), ())), preferred_element_type=jnp.float32)


BM_A = 512
BN_A = 512


def _shared_body(f0, f1, f2, w, b, wg, bg, h32, ti, tw, logit_acc):
    j = pl.program_id(1)
    acc = _dot(f0[...], w[0:D0, :])
    acc += _dot(f1[...], w[D0:D0 + D1, :])
    acc += _dot(f2[...], w[D0 + D1:D0 + D1 + D2, :])
    acc = acc + b[...]
    h = _gelu_exact(acc)
    h32[...] = h
    lg = _dot(h.astype(jnp.bfloat16), wg[...])

    @pl.when(j == 0)
    def _():
        logit_acc[...] = lg

    @pl.when(j != 0)
    def _():
        logit_acc[...] += lg

    @pl.when(j == pl.num_programs(1) - 1)
    def _():
        logits = logit_acc[...] + bg[...]
        m = jnp.max(logits, axis=1, keepdims=True)
        ex = jnp.exp(logits - m)
        p = ex / jnp.sum(ex, axis=1, keepdims=True)
        lane = jax.lax.broadcasted_iota(jnp.int32, p.shape, 1)
        i1 = jnp.argmax(p, axis=1)[:, None].astype(jnp.int32)
        w1 = jnp.max(p, axis=1, keepdims=True)
        p2 = jnp.where(lane == i1, -1.0, p)
        i2 = jnp.argmax(p2, axis=1)[:, None].astype(jnp.int32)
        w2 = jnp.max(p2, axis=1, keepdims=True)
        ti[...] = jnp.concatenate([i1, i2], axis=1)
        tw[...] = jnp.concatenate([w1, w2], axis=1)


def _shared_proj(f0, f1, f2, W16, b_shared, Wg16, b_gate):
    grid = (B // BM_A, F // BN_A)
    return pl.pallas_call(
        _shared_body,
        grid=grid,
        in_specs=[
            pl.BlockSpec((BM_A, D0), lambda i, j: (i, 0)),
            pl.BlockSpec((BM_A, D1), lambda i, j: (i, 0)),
            pl.BlockSpec((BM_A, D2), lambda i, j: (i, 0)),
            pl.BlockSpec((D0 + D1 + D2, BN_A), lambda i, j: (0, j)),
            pl.BlockSpec((1, BN_A), lambda i, j: (0, j)),
            pl.BlockSpec((BN_A, E), lambda i, j: (j, 0)),
            pl.BlockSpec((1, E), lambda i, j: (0, 0)),
        ],
        out_specs=[
            pl.BlockSpec((BM_A, BN_A), lambda i, j: (i, j)),
            pl.BlockSpec((BM_A, TOPK), lambda i, j: (i, 0)),
            pl.BlockSpec((BM_A, TOPK), lambda i, j: (i, 0)),
        ],
        out_shape=[
            jax.ShapeDtypeStruct((B, F), jnp.float32),
            jax.ShapeDtypeStruct((B, TOPK), jnp.int32),
            jax.ShapeDtypeStruct((B, TOPK), jnp.float32),
        ],
        scratch_shapes=[pltpu.VMEM((BM_A, E), jnp.float32)],
        compiler_params=pltpu.CompilerParams(
            dimension_semantics=("parallel", "arbitrary")),
    )(f0, f1, f2, W16, b_shared.reshape(1, F), Wg16, b_gate.reshape(1, E))


@functools.cache
def _route_gather_kernel():
    mesh = plsc.VectorSubcoreMesh(
        core_axis_name="c", subcore_axis_name="s",
        num_cores=NC, num_subcores=NS)

    @functools.partial(
        pl.kernel,
        out_type=[
            jax.ShapeDtypeStruct((NR, F), jnp.float32),
            jax.ShapeDtypeStruct((NR,), jnp.float32),
            jax.ShapeDtypeStruct((A,), jnp.int32),
            jax.ShapeDtypeStruct((64,), jnp.int32),
        ],
        mesh=mesh,
        scratch_types=[
            pltpu.VMEM((SEG,), jnp.int32),
            pltpu.VMEM((SEG,), jnp.float32),
            pltpu.VMEM((SEG,), jnp.int32),
            pltpu.VMEM((SEG,), jnp.int32),
            pltpu.VMEM((16,), jnp.int32),
            pltpu.VMEM((16, 16), jnp.int32),
            pltpu.VMEM((NS, SEG), jnp.int32),
            pltpu.VMEM((NS, SEG), jnp.int32),
            pltpu.VMEM((NS, SEG), jnp.float32),
            pltpu.VMEM((ZSEG,), jnp.int32),
            pltpu.VMEM((ZSEG,), jnp.float32),
            pltpu.VMEM((64,), jnp.int32),
            pltpu.VMEM((GCH, F), jnp.float32),
            pltpu.VMEM((GCH, F), jnp.float32),
            pltpu.VMEM_SHARED((16, 16), jnp.int32),
            pltpu.VMEM_SHARED((NS, SEG), jnp.int32),
            pltpu.VMEM_SHARED((NS, SEG), jnp.int32),
            pltpu.VMEM_SHARED((NS, SEG), jnp.float32),
            pltpu.SemaphoreType.DMA,
            pltpu.SemaphoreType.DMA,
            pltpu.SemaphoreType.DMA,
            pltpu.SemaphoreType.DMA,
        ],
        compiler_params=pltpu.CompilerParams(needs_layout_passes=False),
    )
    def route_gather(tik, twk, h32, hg, wrow_o, pos_o, gid_o,
                     ti_v, tw_v, pos_v, tok_v, cnt_st, cnt_all,
                     pall, tall, wall, acc_i, acc_f, gid_v, bufA, bufB,
                     cnt_sh, all_pos, all_tok, all_tw,
                     semG0, semG1, semW0, semW1):
        s = lax.axis_index("s")
        c = lax.axis_index("c")
        lane = lax.iota(jnp.int32, 16)
        zi16 = jnp.zeros((16,), jnp.int32)
        zf16 = jnp.zeros((16,), jnp.float32)

        def bc(x):
            return jnp.broadcast_to(jnp.asarray(x, jnp.int32), (16,))

        pltpu.sync_copy(tik.at[pl.ds(s * SEG, SEG)], ti_v)
        pltpu.sync_copy(twk.at[pl.ds(s * SEG, SEG)], tw_v)

        cnt = zi16
        for i in range(SEG // 16):
            v = ti_v[pl.ds(i * 16, 16)]
            for e in range(E):
                pc = jnp.sum((v == bc(e)).astype(jnp.int32))
                cnt = cnt + jnp.where(lane == bc(e), bc(pc), zi16)
        cnt_st[...] = cnt
        pltpu.sync_copy(cnt_st, cnt_sh.at[s])
        plsc.subcore_barrier()

        pltpu.sync_copy(cnt_sh, cnt_all)
        pre = zi16
        tot = zi16
        for w in range(NS):
            row = cnt_all[w]
            tot = tot + row
            wmask = bc((jnp.int32(w) < s).astype(jnp.int32))
            pre = pre + wmask * row
        padded = (tot + bc(BLK - 1)) & bc(~(BLK - 1))
        padded = jnp.where(lane < bc(E), padded, zi16)
        incl = plsc.cumsum(padded)
        excl = incl - padded
        base = excl + pre
        total_padded = jnp.sum(jnp.where(lane == bc(E - 1), incl, zi16))
        nb = total_padded // BLK
        exc_s = [jnp.sum(jnp.where(lane == bc(e), excl, zi16)) for e in range(E)]
        pad_s = [jnp.sum(jnp.where(lane == bc(e), padded, zi16)) for e in range(E)]

        cur = base
        for i in range(SEG // 16):
            v = ti_v[pl.ds(i * 16, 16)]
            avec = bc(s * SEG + i * 16) + lane
            tok = avec & bc(B - 1)
            posv = zi16
            for e in range(E):
                mi = (v == bc(e)).astype(jnp.int32)
                pm = plsc.cumsum(mi)
                curE = jnp.sum(jnp.where(lane == bc(e), cur, zi16))
                posv = posv + mi * (bc(curE) + pm - bc(1))
                pc = jnp.sum(mi)
                cur = cur + jnp.where(lane == bc(e), bc(pc), zi16)
            pos_v[pl.ds(i * 16, 16)] = posv
            tok_v[pl.ds(i * 16, 16)] = tok
        pltpu.sync_copy(pos_v, all_pos.at[s])
        pltpu.sync_copy(tok_v, all_tok.at[s])
        pltpu.sync_copy(tw_v, all_tw.at[s])
        plsc.subcore_barrier()

        for w in range(NS):
            pltpu.async_copy(all_pos.at[w], pall.at[w], semG0)
            pltpu.async_copy(all_tok.at[w], tall.at[w], semG0)
            pltpu.async_copy(all_tw.at[w], wall.at[w], semG1)
        for w in range(NS):
            pltpu.make_async_copy(all_pos.at[w], pall.at[w], semG0).wait()
            pltpu.make_async_copy(all_tok.at[w], tall.at[w], semG0).wait()
            pltpu.make_async_copy(all_tw.at[w], wall.at[w], semG1).wait()
        for i in range(ZSEG // 16):
            acc_i[pl.ds(i * 16, 16)] = zi16
            acc_f[pl.ds(i * 16, 16)] = zf16
        w0 = s * ZSEG

        def _scatw(w, _):
            for i in range(SEG // 16):
                pv = pall[w, pl.ds(i * 16, 16)]
                local = pv - bc(w0)
                m = jnp.logical_and(local >= zi16, local < bc(ZSEG))
                idx = jnp.where(m, local, zi16)
                plsc.store_scatter(acc_i, [idx],
                                   tall[w, pl.ds(i * 16, 16)], mask=m)
                plsc.store_scatter(acc_f, [idx],
                                   wall[w, pl.ds(i * 16, 16)], mask=m)
            return 0
        lax.fori_loop(0, NS, _scatw, 0)

        @pl.when(c == 0)
        def _():
            pltpu.sync_copy(pos_v, pos_o.at[pl.ds(s * SEG, SEG)])
            pltpu.sync_copy(acc_f, wrow_o.at[pl.ds(s * ZSEG, ZSEG)])

        @pl.when(jnp.logical_and(c == 0, s == 0))
        def _():
            for vb in range(4):
                jb = (lane + bc(vb * 16)) * bc(BLK)
                g = zi16
                for e in range(E):
                    inblk = jnp.logical_and(jb >= bc(exc_s[e]),
                                            jb < bc(exc_s[e] + pad_s[e]))
                    g = g + bc(e) * inblk.astype(jnp.int32)
                if vb == 3:
                    g = g + jnp.where(lane == bc(15), bc(nb), zi16)
                gid_v[pl.ds(vb * 16, 16)] = g
            pltpu.sync_copy(gid_v, gid_o)

        gbase = s * ZSEG + c * GROWS
        goff = c * GROWS
        nch = GROWS // GCH
        bufs = [bufA, bufB]
        gsems = [semG0, semG1]
        wsems = [semW0, semW1]

        def inr(ch):
            return gbase + ch * GCH < total_padded

        def fire_g(ch):
            @pl.when(inr(ch))
            def _():
                pltpu.async_copy(
                    h32.at[acc_i.at[pl.ds(goff + ch * GCH, GCH)]],
                    bufs[ch % 2], gsems[ch % 2])

        def wait_g(ch):
            @pl.when(inr(ch))
            def _():
                pltpu.make_async_copy(
                    h32.at[acc_i.at[pl.ds(goff + ch * GCH, GCH)]],
                    bufs[ch % 2], gsems[ch % 2]).wait()

        def fire_w(ch):
            @pl.when(inr(ch))
            def _():
                pltpu.async_copy(bufs[ch % 2],
                                 hg.at[pl.ds(gbase + ch * GCH, GCH)],
                                 wsems[ch % 2])

        def wait_w(ch):
            @pl.when(inr(ch))
            def _():
                pltpu.make_async_copy(bufs[ch % 2],
                                      hg.at[pl.ds(gbase + ch * GCH, GCH)],
                                      wsems[ch % 2]).wait()

        fire_g(0)
        for ch in range(nch):
            wait_g(ch)
            if ch + 1 < nch:
                if ch - 1 >= 0:
                    wait_w(ch - 1)
                fire_g(ch + 1)
            fire_w(ch)
        wait_w(nch - 2)
        wait_w(nch - 1)

    return route_gather


def _grouped_body(gid_ref, hg, wexp, bexp, wr, y):
    j = pl.program_id(0)
    nb = gid_ref[63]

    @pl.when(j < nb)
    def _():
        acc = _dot(hg[...].astype(jnp.bfloat16), wexp[0]) + bexp[0]
        y[...] = wr[...] * _gelu_exact(acc)


def _grouped_experts(gid, hg, W16, b_exp3, wrow2):
    grid_spec = pltpu.PrefetchScalarGridSpec(
        num_scalar_prefetch=1,
        grid=(NBLK,),
        in_specs=[
            pl.BlockSpec((BLK, F), lambda j, g: (j, 0)),
            pl.BlockSpec((1, F, F), lambda j, g: (g[j], 0, 0)),
            pl.BlockSpec((1, 1, F), lambda j, g: (g[j], 0, 0)),
            pl.BlockSpec((BLK, 1), lambda j, g: (j, 0)),
        ],
        out_specs=pl.BlockSpec((BLK, F), lambda j, g: (j, 0)),
    )
    return pl.pallas_call(
        _grouped_body,
        grid_spec=grid_spec,
        out_shape=jax.ShapeDtypeStruct((NR, F), jnp.float32),
        compiler_params=pltpu.CompilerParams(
            dimension_semantics=("arbitrary",)),
    )(gid, hg, W16, b_exp3, wrow2)


@functools.cache
def _combine_kernel():
    mesh = plsc.VectorSubcoreMesh(
        core_axis_name="c", subcore_axis_name="s",
        num_cores=NC, num_subcores=NS)

    @functools.partial(
        pl.kernel,
        out_type=jax.ShapeDtypeStruct((B, F), jnp.float32),
        mesh=mesh,
        scratch_types=[
            pltpu.VMEM((TB,), jnp.int32),
            pltpu.VMEM((TB,), jnp.int32),
            pltpu.VMEM((CC, F), jnp.float32),
            pltpu.VMEM((CC, F), jnp.float32),
            pltpu.VMEM((CC, F), jnp.float32),
            pltpu.VMEM((CC, F), jnp.float32),
            pltpu.SemaphoreType.DMA,
            pltpu.SemaphoreType.DMA,
            pltpu.SemaphoreType.DMA,
            pltpu.SemaphoreType.DMA,
        ],
        compiler_params=pltpu.CompilerParams(needs_layout_passes=False),
    )
    def combine(pos_hbm, y_hbm, out_hbm, pos0_l, pos1_l,
                bufA0, bufB0, bufA1, bufB1, semG0, semG1, semW0, semW1):
        s = lax.axis_index("s")
        c = lax.axis_index("c")
        wid = s * NC + c
        t0 = wid * TB
        pltpu.sync_copy(pos_hbm.at[pl.ds(t0, TB)], pos0_l)
        pltpu.sync_copy(pos_hbm.at[pl.ds(B + t0, TB)], pos1_l)
        nch = TB // CC
        bufA = [bufA0, bufA1]
        bufB = [bufB0, bufB1]
        gsems = [semG0, semG1]
        wsems = [semW0, semW1]

        def fire_g(ch):
            p = ch % 2
            pltpu.async_copy(y_hbm.at[pos0_l.at[pl.ds(ch * CC, CC)]],
                             bufA[p], gsems[p])
            pltpu.async_copy(y_hbm.at[pos1_l.at[pl.ds(ch * CC, CC)]],
                             bufB[p], gsems[p])

        def wait_g(ch):
            p = ch % 2
            pltpu.make_async_copy(y_hbm.at[pos0_l.at[pl.ds(ch * CC, CC)]],
                                  bufA[p], gsems[p]).wait()
            pltpu.make_async_copy(y_hbm.at[pos1_l.at[pl.ds(ch * CC, CC)]],
                                  bufB[p], gsems[p]).wait()

        def fire_w(ch):
            p = ch % 2
            pltpu.async_copy(bufA[p], out_hbm.at[pl.ds(t0 + ch * CC, CC)],
                             wsems[p])

        def wait_w(ch):
            p = ch % 2
            pltpu.make_async_copy(bufA[p], out_hbm.at[pl.ds(t0 + ch * CC, CC)],
                                  wsems[p]).wait()

        fire_g(0)
        for ch in range(nch):
            p = ch % 2
            wait_g(ch)
            if ch + 1 < nch:
                if ch - 1 >= 0:
                    wait_w(ch - 1)
                fire_g(ch + 1)

            def _add(i, _):
                off = i * 16
                for r in range(CC):
                    bufA[p][r, pl.ds(off, 16)] += bufB[p][r, pl.ds(off, 16)]
                return 0
            lax.fori_loop(0, F // 16, _add, 0)
            fire_w(ch)
        wait_w(nch - 2)
        wait_w(nch - 1)

    return combine


def kernel(feat0, feat1, feat2, W_shared, b_shared, W_gate, b_gate, W_exp, b_exp):
    bf = jnp.bfloat16
    h32, ti, tw = _shared_proj(feat0.astype(bf), feat1.astype(bf),
                               feat2.astype(bf), W_shared.astype(bf), b_shared,
                               W_gate.astype(bf), b_gate)
    tik = ti.T.reshape(A)
    twk = tw.T.reshape(A)
    hg, wrow, pos, gid = _route_gather_kernel()(tik, twk, h32)
    y = _grouped_experts(gid, hg, W_exp.astype(bf),
                         b_exp.reshape(E, 1, F), wrow.reshape(NR, 1))
    return _combine_kernel()(pos, y)
